# baseline (speedup 1.0000x reference)
import functools

import jax
import jax.numpy as jnp
from jax import lax
from jax.experimental import pallas as pl
from jax.experimental.pallas import tpu as pltpu
from jax.experimental.pallas import tpu_sc as plsc

N = 10000
IN = 128
F = 16
E = 320000
NC = 2
NS = 16
NG = 8
NQ = 4
EQ = E // NQ
CH = 8000
NCH = EQ // CH
L = 16
NVEC = N // L
CVEC = CH // L

_mesh = lambda: plsc.VectorSubcoreMesh(core_axis_name="c", subcore_axis_name="s")
_sc_params = pltpu.CompilerParams(needs_layout_passes=False)


def _tile_ids():
    c = lax.axis_index("c")
    s = lax.axis_index("s")
    g = lax.rem(s, NG)
    q = c * 2 + lax.div(s, NG)
    return g, q


def _zero_plane(acc_v):
    @plsc.parallel_loop(0, NVEC, 1, unroll=8)
    def _(i):
        acc_v[pl.ds(i * L, L)] = jnp.zeros((L,), jnp.float32)


def _edge_pass(q, pk_hbm, vals_hbm, h0_v, h1_v, a0_v, a1_v,
               pk_v, vals_v, sems):
    def start(k, b):
        off = q * EQ + k * CH
        return (pltpu.async_copy(pk_hbm.at[pl.ds(off, CH)], pk_v[b], sems[b]),
                pltpu.async_copy(vals_hbm.at[pl.ds(off, CH)], vals_v[b], sems[b]))

    pending = {0: start(0, 0)}
    for k in range(NCH):
        b = k % 2
        if k + 1 < NCH:
            pending[k + 1] = start(k + 1, (k + 1) % 2)
        for d in pending.pop(k):
            d.wait()

        pv, vv_ = pk_v[b], vals_v[b]

        @plsc.parallel_loop(0, CVEC, 1, unroll=16)
        def _(i):
            d = pl.ds(i * L, L)
            pk = pv[d]
            ri = lax.shift_right_logical(pk, 16)
            ci = lax.bitwise_and(pk, jnp.int32(0xFFFF))
            vv = vv_[d]
            plsc.addupdate_scatter(a0_v, [ri], plsc.load_gather(h0_v, [ci]) * vv)
            plsc.addupdate_scatter(a1_v, [ri], plsc.load_gather(h1_v, [ci]) * vv)


def _make_spmm_plain():
    @functools.partial(
        pl.kernel,
        out_type=jax.ShapeDtypeStruct((NQ * F * N,), jnp.float32),
        mesh=_mesh(),
        compiler_params=_sc_params,
        scratch_types=[
            pltpu.VMEM((N,), jnp.float32),
            pltpu.VMEM((N,), jnp.float32),
            pltpu.VMEM((N,), jnp.float32),
            pltpu.VMEM((N,), jnp.float32),
            pltpu.VMEM((CH,), jnp.int32),
            pltpu.VMEM((CH,), jnp.int32),
            pltpu.VMEM((CH,), jnp.float32),
            pltpu.VMEM((CH,), jnp.float32),
            pltpu.SemaphoreType.DMA,
            pltpu.SemaphoreType.DMA,
        ],
    )
    def spmm(h_hbm, pk_hbm, vals_hbm, out_hbm,
             h0_v, h1_v, a0_v, a1_v, p0, p1, v0, v1, sem0, sem1):
        g, q = _tile_ids()
        f0 = 2 * g
        d0 = pltpu.async_copy(h_hbm.at[pl.ds(f0 * N, N)], h0_v, sem0)
        d1 = pltpu.async_copy(h_hbm.at[pl.ds((f0 + 1) * N, N)], h1_v, sem0)
        d0.wait(); d1.wait()
        _zero_plane(a0_v)
        _zero_plane(a1_v)
        _edge_pass(q, pk_hbm, vals_hbm, h0_v, h1_v, a0_v, a1_v,
                   (p0, p1), (v0, v1), (sem0, sem1))
        e0 = pltpu.async_copy(a0_v, out_hbm.at[pl.ds((q * F + f0) * N, N)], sem0)
        e1 = pltpu.async_copy(a1_v, out_hbm.at[pl.ds((q * F + f0 + 1) * N, N)], sem0)
        e0.wait(); e1.wait()

    return spmm


def _make_spmm_merge():
    @functools.partial(
        pl.kernel,
        out_type=jax.ShapeDtypeStruct((NQ * F * N,), jnp.float32),
        mesh=_mesh(),
        compiler_params=_sc_params,
        scratch_types=[
            pltpu.VMEM((N,), jnp.float32),
            pltpu.VMEM((N,), jnp.float32),
            pltpu.VMEM((N,), jnp.float32),
            pltpu.VMEM((N,), jnp.float32),
            pltpu.VMEM((N,), jnp.float32),
            pltpu.VMEM((N,), jnp.float32),
            pltpu.VMEM((N,), jnp.float32),
            pltpu.VMEM((N,), jnp.float32),
            pltpu.VMEM((N,), jnp.float32),
            pltpu.VMEM((CH,), jnp.int32),
            pltpu.VMEM((CH,), jnp.int32),
            pltpu.VMEM((CH,), jnp.float32),
            pltpu.VMEM((CH,), jnp.float32),
            pltpu.SemaphoreType.DMA,
            pltpu.SemaphoreType.DMA,
        ],
    )
    def spmm(prev_hbm, f_hbm, pk_hbm, vals_hbm, out_hbm,
             b00, b01, b02, b03, b10, b11, b12, b13, fv_v,
             p0, p1, v0, v1, sem0, sem1):
        g, q = _tile_ids()
        f0 = 2 * g
        ds_ = []
        for qq, dst in ((0, b00), (1, b01), (2, b02), (3, b03)):
            ds_.append(pltpu.async_copy(
                prev_hbm.at[pl.ds((qq * F + f0) * N, N)], dst, sem0))
        for qq, dst in ((0, b10), (1, b11), (2, b12), (3, b13)):
            ds_.append(pltpu.async_copy(
                prev_hbm.at[pl.ds((qq * F + f0 + 1) * N, N)], dst, sem0))
        ds_.append(pltpu.async_copy(f_hbm, fv_v, sem0))
        for d in ds_:
            d.wait()

        @plsc.parallel_loop(0, NVEC, 1, unroll=8)
        def _(i):
            d = pl.ds(i * L, L)
            fv = fv_v[d]
            b00[d] = (b00[d] + b01[d] + b02[d] + b03[d]) * fv
            b10[d] = (b10[d] + b11[d] + b12[d] + b13[d]) * fv

        h0_v, h1_v, a0_v, a1_v = b00, b10, b01, b11
        _zero_plane(a0_v)
        _zero_plane(a1_v)
        _edge_pass(q, pk_hbm, vals_hbm, h0_v, h1_v, a0_v, a1_v,
                   (p0, p1), (v0, v1), (sem0, sem1))
        e0 = pltpu.async_copy(a0_v, out_hbm.at[pl.ds((q * F + f0) * N, N)], sem0)
        e1 = pltpu.async_copy(a1_v, out_hbm.at[pl.ds((q * F + f0 + 1) * N, N)], sem0)
        e0.wait(); e1.wait()

    return spmm


def _make_final_gather(nb):
    half = nb // NC

    @functools.partial(
        pl.kernel,
        out_type=jax.ShapeDtypeStruct((F * nb,), jnp.float32),
        mesh=_mesh(),
        compiler_params=_sc_params,
        scratch_types=[
            pltpu.VMEM((N,), jnp.float32),
            pltpu.VMEM((N,), jnp.float32),
            pltpu.VMEM((N,), jnp.float32),
            pltpu.VMEM((N,), jnp.float32),
            pltpu.VMEM((half,), jnp.int32),
            pltpu.VMEM((half,), jnp.float32),
            pltpu.SemaphoreType.DMA,
        ],
    )
    def fin(prev_hbm, idx_hbm, out_hbm, t0, t1, t2, t3, idx_v, ob_v, sem0):
        c = lax.axis_index("c")
        s = lax.axis_index("s")
        ds_ = [pltpu.async_copy(prev_hbm.at[pl.ds((qq * F + s) * N, N)], dst, sem0)
               for qq, dst in ((0, t0), (1, t1), (2, t2), (3, t3))]
        ds_.append(pltpu.async_copy(idx_hbm.at[pl.ds(c * half, half)], idx_v, sem0))
        for d in ds_:
            d.wait()

        @plsc.parallel_loop(0, NVEC, 1, unroll=8)
        def _(i):
            d = pl.ds(i * L, L)
            t0[d] = t0[d] + t1[d] + t2[d] + t3[d]

        @plsc.parallel_loop(0, half // L, 1, unroll=8)
        def _(i):
            d = pl.ds(i * L, L)
            ob_v[d] = plsc.load_gather(t0, [idx_v[d]])

        pltpu.sync_copy(ob_v, out_hbm.at[pl.ds(s * nb + c * half, half)])

    return fin


def _tc_mid_matmul(parts, W2):
    def body(p_ref, w_ref, o_ref):
        h2 = jax.nn.relu(p_ref[0] + p_ref[1] + p_ref[2] + p_ref[3])
        o_ref[...] = lax.dot_general(
            w_ref[...], h2, (((0,), (0,)), ((), ())),
            preferred_element_type=jnp.float32)

    return pl.pallas_call(
        body,
        out_shape=jax.ShapeDtypeStruct((F, N), jnp.float32),
    )(parts, W2)


def _tc_prep(x, W1, w_idx, i_idx):
    def body(x_ref, w_ref, wi_ref, ii_ref, h_ref, wp_ref, ip_ref):
        xt = x_ref[...].T
        h_ref[...] = lax.dot_general(
            w_ref[...], xt, (((0,), (0,)), ((), ())),
            preferred_element_type=jnp.float32)
        wp_ref[...] = lax.shift_left(wi_ref[0], 16) | wi_ref[1]
        ip_ref[...] = lax.shift_left(ii_ref[0], 16) | ii_ref[1]

    h0t, w_pk, i_pk = pl.pallas_call(
        body,
        out_shape=[
            jax.ShapeDtypeStruct((F, N), jnp.float32),
            jax.ShapeDtypeStruct((E // 128, 128), jnp.int32),
            jax.ShapeDtypeStruct((E // 128, 128), jnp.int32),
        ],
    )(x, W1, w_idx.reshape(2, E // 128, 128), i_idx.reshape(2, E // 128, 128))
    return h0t, w_pk.reshape(-1), i_pk.reshape(-1)


@jax.jit
def kernel(x, wavelet_indices, wavelet_values, inverse_wavelet_indices,
           inverse_wavelet_values, index, W1, f1, W2, f2):
    h0t, w_pk, i_pk = _tc_prep(x, W1, wavelet_indices, inverse_wavelet_indices)
    w_vals = wavelet_values
    i_vals = inverse_wavelet_values

    spmm_plain = _make_spmm_plain()
    spmm_merge = _make_spmm_merge()
    fin = _make_final_gather(index.shape[0])

    nb = index.shape[0]
    pA = spmm_plain(h0t.reshape(-1), i_pk, i_vals)
    pB = spmm_merge(pA, f1, w_pk, w_vals)
    h3t = _tc_mid_matmul(pB.reshape(NQ, F, N), W2)
    pC = spmm_plain(h3t.reshape(-1), i_pk, i_vals)
    pD = spmm_merge(pC, f2, w_pk, w_vals)
    out_t = fin(pD, index)
    return out_t.reshape(F, nb).T


# speedup vs baseline: 1.0395x; 1.0395x over previous
import functools

import jax
import jax.numpy as jnp
from jax import lax
from jax.experimental import pallas as pl
from jax.experimental.pallas import tpu as pltpu
from jax.experimental.pallas import tpu_sc as plsc

N = 10000
IN = 128
F = 16
E = 320000
NC = 2
NS = 16
NG = 8
NQ = 4
EQ = E // NQ
CH = 8000
NCH = EQ // CH
L = 16
NVEC = N // L
CVEC = CH // L

_mesh = lambda: plsc.VectorSubcoreMesh(core_axis_name="c", subcore_axis_name="s")
_sc_params = pltpu.CompilerParams(needs_layout_passes=False)


def _tile_ids():
    c = lax.axis_index("c")
    s = lax.axis_index("s")
    g = lax.rem(s, NG)
    q = c * 2 + lax.div(s, NG)
    return g, q


def _zero_plane(acc_v):
    @plsc.parallel_loop(0, NVEC, 1, unroll=8)
    def _(i):
        acc_v[pl.ds(i * L, L)] = jnp.zeros((L,), jnp.float32)


def _edge_pass(q, pk_hbm, vals_hbm, h0_v, h1_v, a0_v, a1_v,
               pk_v, vals_v, sems):
    def start(k, b):
        off = q * EQ + k * CH
        return (pltpu.async_copy(pk_hbm.at[pl.ds(off, CH)], pk_v[b], sems[b]),
                pltpu.async_copy(vals_hbm.at[pl.ds(off, CH)], vals_v[b], sems[b]))

    pending = {0: start(0, 0)}
    for k in range(NCH):
        b = k % 2
        if k + 1 < NCH:
            pending[k + 1] = start(k + 1, (k + 1) % 2)
        for d in pending.pop(k):
            d.wait()

        pv, vv_ = pk_v[b], vals_v[b]

        @plsc.parallel_loop(0, CVEC, 1, unroll=8)
        def _(i):
            d = pl.ds(i * L, L)
            pk = pv[d]
            ri = lax.shift_right_logical(pk, 16)
            ci = lax.bitwise_and(pk, jnp.int32(0xFFFF))
            vv = vv_[d]
            plsc.addupdate_scatter(a0_v, [ri], plsc.load_gather(h0_v, [ci]) * vv)
            plsc.addupdate_scatter(a1_v, [ri], plsc.load_gather(h1_v, [ci]) * vv)


def _make_spmm_plain():
    @functools.partial(
        pl.kernel,
        out_type=jax.ShapeDtypeStruct((NQ * F * N,), jnp.float32),
        mesh=_mesh(),
        compiler_params=_sc_params,
        scratch_types=[
            pltpu.VMEM((N,), jnp.float32),
            pltpu.VMEM((N,), jnp.float32),
            pltpu.VMEM((N,), jnp.float32),
            pltpu.VMEM((N,), jnp.float32),
            pltpu.VMEM((CH,), jnp.int32),
            pltpu.VMEM((CH,), jnp.int32),
            pltpu.VMEM((CH,), jnp.float32),
            pltpu.VMEM((CH,), jnp.float32),
            pltpu.SemaphoreType.DMA,
            pltpu.SemaphoreType.DMA,
        ],
    )
    def spmm(h_hbm, pk_hbm, vals_hbm, out_hbm,
             h0_v, h1_v, a0_v, a1_v, p0, p1, v0, v1, sem0, sem1):
        g, q = _tile_ids()
        f0 = 2 * g
        d0 = pltpu.async_copy(h_hbm.at[pl.ds(f0 * N, N)], h0_v, sem0)
        d1 = pltpu.async_copy(h_hbm.at[pl.ds((f0 + 1) * N, N)], h1_v, sem0)
        d0.wait(); d1.wait()
        _zero_plane(a0_v)
        _zero_plane(a1_v)
        _edge_pass(q, pk_hbm, vals_hbm, h0_v, h1_v, a0_v, a1_v,
                   (p0, p1), (v0, v1), (sem0, sem1))
        e0 = pltpu.async_copy(a0_v, out_hbm.at[pl.ds((q * F + f0) * N, N)], sem0)
        e1 = pltpu.async_copy(a1_v, out_hbm.at[pl.ds((q * F + f0 + 1) * N, N)], sem0)
        e0.wait(); e1.wait()

    return spmm


def _make_spmm_merge():
    @functools.partial(
        pl.kernel,
        out_type=jax.ShapeDtypeStruct((NQ * F * N,), jnp.float32),
        mesh=_mesh(),
        compiler_params=_sc_params,
        scratch_types=[
            pltpu.VMEM((N,), jnp.float32),
            pltpu.VMEM((N,), jnp.float32),
            pltpu.VMEM((N,), jnp.float32),
            pltpu.VMEM((N,), jnp.float32),
            pltpu.VMEM((N,), jnp.float32),
            pltpu.VMEM((N,), jnp.float32),
            pltpu.VMEM((N,), jnp.float32),
            pltpu.VMEM((N,), jnp.float32),
            pltpu.VMEM((N,), jnp.float32),
            pltpu.VMEM((CH,), jnp.int32),
            pltpu.VMEM((CH,), jnp.int32),
            pltpu.VMEM((CH,), jnp.float32),
            pltpu.VMEM((CH,), jnp.float32),
            pltpu.SemaphoreType.DMA,
            pltpu.SemaphoreType.DMA,
        ],
    )
    def spmm(prev_hbm, f_hbm, pk_hbm, vals_hbm, out_hbm,
             b00, b01, b02, b03, b10, b11, b12, b13, fv_v,
             p0, p1, v0, v1, sem0, sem1):
        g, q = _tile_ids()
        f0 = 2 * g
        ds_ = []
        for qq, dst in ((0, b00), (1, b01), (2, b02), (3, b03)):
            ds_.append(pltpu.async_copy(
                prev_hbm.at[pl.ds((qq * F + f0) * N, N)], dst, sem0))
        for qq, dst in ((0, b10), (1, b11), (2, b12), (3, b13)):
            ds_.append(pltpu.async_copy(
                prev_hbm.at[pl.ds((qq * F + f0 + 1) * N, N)], dst, sem0))
        ds_.append(pltpu.async_copy(f_hbm, fv_v, sem0))
        for d in ds_:
            d.wait()

        @plsc.parallel_loop(0, NVEC, 1, unroll=8)
        def _(i):
            d = pl.ds(i * L, L)
            fv = fv_v[d]
            b00[d] = (b00[d] + b01[d] + b02[d] + b03[d]) * fv
            b10[d] = (b10[d] + b11[d] + b12[d] + b13[d]) * fv

        h0_v, h1_v, a0_v, a1_v = b00, b10, b01, b11
        _zero_plane(a0_v)
        _zero_plane(a1_v)
        _edge_pass(q, pk_hbm, vals_hbm, h0_v, h1_v, a0_v, a1_v,
                   (p0, p1), (v0, v1), (sem0, sem1))
        e0 = pltpu.async_copy(a0_v, out_hbm.at[pl.ds((q * F + f0) * N, N)], sem0)
        e1 = pltpu.async_copy(a1_v, out_hbm.at[pl.ds((q * F + f0 + 1) * N, N)], sem0)
        e0.wait(); e1.wait()

    return spmm


def _make_final_gather(nb):
    half = nb // NC

    @functools.partial(
        pl.kernel,
        out_type=jax.ShapeDtypeStruct((F * nb,), jnp.float32),
        mesh=_mesh(),
        compiler_params=_sc_params,
        scratch_types=[
            pltpu.VMEM((N,), jnp.float32),
            pltpu.VMEM((N,), jnp.float32),
            pltpu.VMEM((N,), jnp.float32),
            pltpu.VMEM((N,), jnp.float32),
            pltpu.VMEM((half,), jnp.int32),
            pltpu.VMEM((half,), jnp.float32),
            pltpu.SemaphoreType.DMA,
        ],
    )
    def fin(prev_hbm, idx_hbm, out_hbm, t0, t1, t2, t3, idx_v, ob_v, sem0):
        c = lax.axis_index("c")
        s = lax.axis_index("s")
        ds_ = [pltpu.async_copy(prev_hbm.at[pl.ds((qq * F + s) * N, N)], dst, sem0)
               for qq, dst in ((0, t0), (1, t1), (2, t2), (3, t3))]
        ds_.append(pltpu.async_copy(idx_hbm.at[pl.ds(c * half, half)], idx_v, sem0))
        for d in ds_:
            d.wait()

        @plsc.parallel_loop(0, NVEC, 1, unroll=8)
        def _(i):
            d = pl.ds(i * L, L)
            t0[d] = t0[d] + t1[d] + t2[d] + t3[d]

        @plsc.parallel_loop(0, half // L, 1, unroll=8)
        def _(i):
            d = pl.ds(i * L, L)
            ob_v[d] = plsc.load_gather(t0, [idx_v[d]])

        pltpu.sync_copy(ob_v, out_hbm.at[pl.ds(s * nb + c * half, half)])

    return fin


def _tc_mid_matmul(parts, W2):
    def body(p_ref, w_ref, o_ref):
        h2 = jax.nn.relu(p_ref[0] + p_ref[1] + p_ref[2] + p_ref[3])
        o_ref[...] = lax.dot_general(
            w_ref[...], h2, (((0,), (0,)), ((), ())),
            preferred_element_type=jnp.float32)

    return pl.pallas_call(
        body,
        out_shape=jax.ShapeDtypeStruct((F, N), jnp.float32),
    )(parts, W2)


def _tc_prep(x, W1, w_idx, i_idx):
    def body(x_ref, w_ref, wi_ref, ii_ref, h_ref, wp_ref, ip_ref):
        xt = x_ref[...].T
        h_ref[...] = lax.dot_general(
            w_ref[...], xt, (((0,), (0,)), ((), ())),
            preferred_element_type=jnp.float32)
        wp_ref[...] = lax.shift_left(wi_ref[0], 16) | wi_ref[1]
        ip_ref[...] = lax.shift_left(ii_ref[0], 16) | ii_ref[1]

    h0t, w_pk, i_pk = pl.pallas_call(
        body,
        out_shape=[
            jax.ShapeDtypeStruct((F, N), jnp.float32),
            jax.ShapeDtypeStruct((E // 128, 128), jnp.int32),
            jax.ShapeDtypeStruct((E // 128, 128), jnp.int32),
        ],
    )(x, W1, w_idx.reshape(2, E // 128, 128), i_idx.reshape(2, E // 128, 128))
    return h0t, w_pk.reshape(-1), i_pk.reshape(-1)


@jax.jit
def kernel(x, wavelet_indices, wavelet_values, inverse_wavelet_indices,
           inverse_wavelet_values, index, W1, f1, W2, f2):
    h0t, w_pk, i_pk = _tc_prep(x, W1, wavelet_indices, inverse_wavelet_indices)
    w_vals = wavelet_values
    i_vals = inverse_wavelet_values

    spmm_plain = _make_spmm_plain()
    spmm_merge = _make_spmm_merge()
    fin = _make_final_gather(index.shape[0])

    nb = index.shape[0]
    pA = spmm_plain(h0t.reshape(-1), i_pk, i_vals)
    pB = spmm_merge(pA, f1, w_pk, w_vals)
    h3t = _tc_mid_matmul(pB.reshape(NQ, F, N), W2)
    pC = spmm_plain(h3t.reshape(-1), i_pk, i_vals)
    pD = spmm_merge(pC, f2, w_pk, w_vals)
    out_t = fin(pD, index)
    return out_t.reshape(F, nb).T


# speedup vs baseline: 1.0704x; 1.0298x over previous
import functools

import jax
import jax.numpy as jnp
from jax import lax
from jax.experimental import pallas as pl
from jax.experimental.pallas import tpu as pltpu
from jax.experimental.pallas import tpu_sc as plsc

N = 10000
IN = 128
F = 16
E = 320000
NC = 2
NS = 16
NG = 8
NQ = 4
EQ = E // NQ
CH = 8000
NCH = EQ // CH
L = 16
NVEC = N // L
CVEC = CH // L

_mesh = lambda: plsc.VectorSubcoreMesh(core_axis_name="c", subcore_axis_name="s")
_sc_params = pltpu.CompilerParams(needs_layout_passes=False)


def _tile_ids():
    c = lax.axis_index("c")
    s = lax.axis_index("s")
    g = lax.rem(s, NG)
    q = c * 2 + lax.div(s, NG)
    return g, q


def _zero_plane(acc_v):
    @plsc.parallel_loop(0, NVEC, 1, unroll=8)
    def _(i):
        acc_v[pl.ds(i * L, L)] = jnp.zeros((L,), jnp.float32)


def _start_chunk(q, k, pk_hbm, vals_hbm, pk_b, vals_b, sem):
    off = q * EQ + k * CH
    return (pltpu.async_copy(pk_hbm.at[pl.ds(off, CH)], pk_b, sem),
            pltpu.async_copy(vals_hbm.at[pl.ds(off, CH)], vals_b, sem))


def _edge_pass(q, pk_hbm, vals_hbm, h0_v, h1_v, a0_v, a1_v,
               pk_v, vals_v, sems, primed):
    def start(k, b):
        return _start_chunk(q, k, pk_hbm, vals_hbm, pk_v[b], vals_v[b], sems[b])

    pending = {0: primed}
    for k in range(NCH):
        b = k % 2
        if k + 1 < NCH:
            pending[k + 1] = start(k + 1, (k + 1) % 2)
        for d in pending.pop(k):
            d.wait()

        pv, vv_ = pk_v[b], vals_v[b]

        @plsc.parallel_loop(0, CVEC, 1, unroll=8)
        def _(i):
            d = pl.ds(i * L, L)
            pk = pv[d]
            ri = lax.shift_right_logical(pk, 16)
            ci = lax.bitwise_and(pk, jnp.int32(0xFFFF))
            vv = vv_[d]
            plsc.addupdate_scatter(a0_v, [ri], plsc.load_gather(h0_v, [ci]) * vv)
            plsc.addupdate_scatter(a1_v, [ri], plsc.load_gather(h1_v, [ci]) * vv)


def _make_spmm_plain():
    @functools.partial(
        pl.kernel,
        out_type=jax.ShapeDtypeStruct((NQ * F * N,), jnp.float32),
        mesh=_mesh(),
        compiler_params=_sc_params,
        scratch_types=[
            pltpu.VMEM((N,), jnp.float32),
            pltpu.VMEM((N,), jnp.float32),
            pltpu.VMEM((N,), jnp.float32),
            pltpu.VMEM((N,), jnp.float32),
            pltpu.VMEM((CH,), jnp.int32),
            pltpu.VMEM((CH,), jnp.int32),
            pltpu.VMEM((CH,), jnp.float32),
            pltpu.VMEM((CH,), jnp.float32),
            pltpu.SemaphoreType.DMA,
            pltpu.SemaphoreType.DMA,
            pltpu.SemaphoreType.DMA,
        ],
    )
    def spmm(h_hbm, pk_hbm, vals_hbm, out_hbm,
             h0_v, h1_v, a0_v, a1_v, p0, p1, v0, v1, sem0, sem1, sem2):
        g, q = _tile_ids()
        f0 = 2 * g
        primed = _start_chunk(q, 0, pk_hbm, vals_hbm, p0, v0, sem0)
        d0 = pltpu.async_copy(h_hbm.at[pl.ds(f0 * N, N)], h0_v, sem2)
        d1 = pltpu.async_copy(h_hbm.at[pl.ds((f0 + 1) * N, N)], h1_v, sem2)
        _zero_plane(a0_v)
        _zero_plane(a1_v)
        d0.wait(); d1.wait()
        _edge_pass(q, pk_hbm, vals_hbm, h0_v, h1_v, a0_v, a1_v,
                   (p0, p1), (v0, v1), (sem0, sem1), primed)
        e0 = pltpu.async_copy(a0_v, out_hbm.at[pl.ds((q * F + f0) * N, N)], sem2)
        e1 = pltpu.async_copy(a1_v, out_hbm.at[pl.ds((q * F + f0 + 1) * N, N)], sem2)
        e0.wait(); e1.wait()

    return spmm


def _make_spmm_merge():
    @functools.partial(
        pl.kernel,
        out_type=jax.ShapeDtypeStruct((NQ * F * N,), jnp.float32),
        mesh=_mesh(),
        compiler_params=_sc_params,
        scratch_types=[
            pltpu.VMEM((N,), jnp.float32),
            pltpu.VMEM((N,), jnp.float32),
            pltpu.VMEM((N,), jnp.float32),
            pltpu.VMEM((N,), jnp.float32),
            pltpu.VMEM((N,), jnp.float32),
            pltpu.VMEM((N,), jnp.float32),
            pltpu.VMEM((N,), jnp.float32),
            pltpu.VMEM((N,), jnp.float32),
            pltpu.VMEM((N,), jnp.float32),
            pltpu.VMEM((CH,), jnp.int32),
            pltpu.VMEM((CH,), jnp.int32),
            pltpu.VMEM((CH,), jnp.float32),
            pltpu.VMEM((CH,), jnp.float32),
            pltpu.SemaphoreType.DMA,
            pltpu.SemaphoreType.DMA,
            pltpu.SemaphoreType.DMA,
        ],
    )
    def spmm(prev_hbm, f_hbm, pk_hbm, vals_hbm, out_hbm,
             b00, b01, b02, b03, b10, b11, b12, b13, fv_v,
             p0, p1, v0, v1, sem0, sem1, sem2):
        g, q = _tile_ids()
        f0 = 2 * g
        primed = _start_chunk(q, 0, pk_hbm, vals_hbm, p0, v0, sem0)
        ds_ = []
        for qq, dst in ((0, b00), (1, b01), (2, b02), (3, b03)):
            ds_.append(pltpu.async_copy(
                prev_hbm.at[pl.ds((qq * F + f0) * N, N)], dst, sem2))
        for qq, dst in ((0, b10), (1, b11), (2, b12), (3, b13)):
            ds_.append(pltpu.async_copy(
                prev_hbm.at[pl.ds((qq * F + f0 + 1) * N, N)], dst, sem2))
        ds_.append(pltpu.async_copy(f_hbm, fv_v, sem2))
        for d in ds_:
            d.wait()

        @plsc.parallel_loop(0, NVEC, 1, unroll=8)
        def _(i):
            d = pl.ds(i * L, L)
            fv = fv_v[d]
            b00[d] = (b00[d] + b01[d] + b02[d] + b03[d]) * fv
            b10[d] = (b10[d] + b11[d] + b12[d] + b13[d]) * fv

        h0_v, h1_v, a0_v, a1_v = b00, b10, b01, b11
        _zero_plane(a0_v)
        _zero_plane(a1_v)
        _edge_pass(q, pk_hbm, vals_hbm, h0_v, h1_v, a0_v, a1_v,
                   (p0, p1), (v0, v1), (sem0, sem1), primed)
        e0 = pltpu.async_copy(a0_v, out_hbm.at[pl.ds((q * F + f0) * N, N)], sem2)
        e1 = pltpu.async_copy(a1_v, out_hbm.at[pl.ds((q * F + f0 + 1) * N, N)], sem2)
        e0.wait(); e1.wait()

    return spmm


def _make_final_gather(nb):
    half = nb // NC

    @functools.partial(
        pl.kernel,
        out_type=jax.ShapeDtypeStruct((F * nb,), jnp.float32),
        mesh=_mesh(),
        compiler_params=_sc_params,
        scratch_types=[
            pltpu.VMEM((N,), jnp.float32),
            pltpu.VMEM((N,), jnp.float32),
            pltpu.VMEM((N,), jnp.float32),
            pltpu.VMEM((N,), jnp.float32),
            pltpu.VMEM((half,), jnp.int32),
            pltpu.VMEM((half,), jnp.float32),
            pltpu.SemaphoreType.DMA,
        ],
    )
    def fin(prev_hbm, idx_hbm, out_hbm, t0, t1, t2, t3, idx_v, ob_v, sem0):
        c = lax.axis_index("c")
        s = lax.axis_index("s")
        ds_ = [pltpu.async_copy(prev_hbm.at[pl.ds((qq * F + s) * N, N)], dst, sem0)
               for qq, dst in ((0, t0), (1, t1), (2, t2), (3, t3))]
        ds_.append(pltpu.async_copy(idx_hbm.at[pl.ds(c * half, half)], idx_v, sem0))
        for d in ds_:
            d.wait()

        @plsc.parallel_loop(0, NVEC, 1, unroll=8)
        def _(i):
            d = pl.ds(i * L, L)
            t0[d] = t0[d] + t1[d] + t2[d] + t3[d]

        @plsc.parallel_loop(0, half // L, 1, unroll=8)
        def _(i):
            d = pl.ds(i * L, L)
            ob_v[d] = plsc.load_gather(t0, [idx_v[d]])

        pltpu.sync_copy(ob_v, out_hbm.at[pl.ds(s * nb + c * half, half)])

    return fin


def _tc_mid_matmul(parts, W2):
    def body(p_ref, w_ref, o_ref):
        h2 = jax.nn.relu(p_ref[0] + p_ref[1] + p_ref[2] + p_ref[3])
        o_ref[...] = lax.dot_general(
            w_ref[...], h2, (((0,), (0,)), ((), ())),
            preferred_element_type=jnp.float32)

    return pl.pallas_call(
        body,
        out_shape=jax.ShapeDtypeStruct((F, N), jnp.float32),
    )(parts, W2)


def _tc_prep(x, W1, w_idx, i_idx):
    def body(x_ref, w_ref, wi_ref, ii_ref, h_ref, wp_ref, ip_ref):
        xt = x_ref[...].T
        h_ref[...] = lax.dot_general(
            w_ref[...], xt, (((0,), (0,)), ((), ())),
            preferred_element_type=jnp.float32)
        wp_ref[...] = lax.shift_left(wi_ref[0], 16) | wi_ref[1]
        ip_ref[...] = lax.shift_left(ii_ref[0], 16) | ii_ref[1]

    h0t, w_pk, i_pk = pl.pallas_call(
        body,
        out_shape=[
            jax.ShapeDtypeStruct((F, N), jnp.float32),
            jax.ShapeDtypeStruct((E // 128, 128), jnp.int32),
            jax.ShapeDtypeStruct((E // 128, 128), jnp.int32),
        ],
    )(x, W1, w_idx.reshape(2, E // 128, 128), i_idx.reshape(2, E // 128, 128))
    return h0t, w_pk.reshape(-1), i_pk.reshape(-1)


@jax.jit
def kernel(x, wavelet_indices, wavelet_values, inverse_wavelet_indices,
           inverse_wavelet_values, index, W1, f1, W2, f2):
    h0t, w_pk, i_pk = _tc_prep(x, W1, wavelet_indices, inverse_wavelet_indices)
    w_vals = wavelet_values
    i_vals = inverse_wavelet_values

    spmm_plain = _make_spmm_plain()
    spmm_merge = _make_spmm_merge()
    fin = _make_final_gather(index.shape[0])

    nb = index.shape[0]
    pA = spmm_plain(h0t.reshape(-1), i_pk, i_vals)
    pB = spmm_merge(pA, f1, w_pk, w_vals)
    h3t = _tc_mid_matmul(pB.reshape(NQ, F, N), W2)
    pC = spmm_plain(h3t.reshape(-1), i_pk, i_vals)
    pD = spmm_merge(pC, f2, w_pk, w_vals)
    out_t = fin(pD, index)
    return out_t.reshape(F, nb).T


# speedup vs baseline: 1.0712x; 1.0008x over previous
import functools

import jax
import jax.numpy as jnp
from jax import lax
from jax.experimental import pallas as pl
from jax.experimental.pallas import tpu as pltpu
from jax.experimental.pallas import tpu_sc as plsc

N = 10000
IN = 128
F = 16
E = 320000
NC = 2
NS = 16
NG = 8
NQ = 4
EQ = E // NQ
CH = 8000
NCH = EQ // CH
L = 16
NVEC = N // L
CVEC = CH // L

_mesh = lambda: plsc.VectorSubcoreMesh(core_axis_name="c", subcore_axis_name="s")
_sc_params = pltpu.CompilerParams(needs_layout_passes=False, skip_device_barrier=True)


def _tile_ids():
    c = lax.axis_index("c")
    s = lax.axis_index("s")
    g = lax.rem(s, NG)
    q = c * 2 + lax.div(s, NG)
    return g, q


def _zero_plane(acc_v):
    @plsc.parallel_loop(0, NVEC, 1, unroll=8)
    def _(i):
        acc_v[pl.ds(i * L, L)] = jnp.zeros((L,), jnp.float32)


def _start_chunk(q, k, pk_hbm, vals_hbm, pk_b, vals_b, sem):
    off = q * EQ + k * CH
    return (pltpu.async_copy(pk_hbm.at[pl.ds(off, CH)], pk_b, sem),
            pltpu.async_copy(vals_hbm.at[pl.ds(off, CH)], vals_b, sem))


def _edge_pass(q, pk_hbm, vals_hbm, h0_v, h1_v, a0_v, a1_v,
               pk_v, vals_v, sems, primed):
    def start(k, b):
        return _start_chunk(q, k, pk_hbm, vals_hbm, pk_v[b], vals_v[b], sems[b])

    pending = {0: primed}
    for k in range(NCH):
        b = k % 2
        if k + 1 < NCH:
            pending[k + 1] = start(k + 1, (k + 1) % 2)
        for d in pending.pop(k):
            d.wait()

        pv, vv_ = pk_v[b], vals_v[b]

        @plsc.parallel_loop(0, CVEC, 1, unroll=8)
        def _(i):
            d = pl.ds(i * L, L)
            pk = pv[d]
            ri = lax.shift_right_logical(pk, 16)
            ci = lax.bitwise_and(pk, jnp.int32(0xFFFF))
            vv = vv_[d]
            plsc.addupdate_scatter(a0_v, [ri], plsc.load_gather(h0_v, [ci]) * vv)
            plsc.addupdate_scatter(a1_v, [ri], plsc.load_gather(h1_v, [ci]) * vv)


def _make_spmm_plain():
    @functools.partial(
        pl.kernel,
        out_type=jax.ShapeDtypeStruct((NQ * F * N,), jnp.float32),
        mesh=_mesh(),
        compiler_params=_sc_params,
        scratch_types=[
            pltpu.VMEM((N,), jnp.float32),
            pltpu.VMEM((N,), jnp.float32),
            pltpu.VMEM((N,), jnp.float32),
            pltpu.VMEM((N,), jnp.float32),
            pltpu.VMEM((CH,), jnp.int32),
            pltpu.VMEM((CH,), jnp.int32),
            pltpu.VMEM((CH,), jnp.float32),
            pltpu.VMEM((CH,), jnp.float32),
            pltpu.SemaphoreType.DMA,
            pltpu.SemaphoreType.DMA,
            pltpu.SemaphoreType.DMA,
        ],
    )
    def spmm(h_hbm, pk_hbm, vals_hbm, out_hbm,
             h0_v, h1_v, a0_v, a1_v, p0, p1, v0, v1, sem0, sem1, sem2):
        g, q = _tile_ids()
        f0 = 2 * g
        primed = _start_chunk(q, 0, pk_hbm, vals_hbm, p0, v0, sem0)
        d0 = pltpu.async_copy(h_hbm.at[pl.ds(f0 * N, N)], h0_v, sem2)
        d1 = pltpu.async_copy(h_hbm.at[pl.ds((f0 + 1) * N, N)], h1_v, sem2)
        _zero_plane(a0_v)
        _zero_plane(a1_v)
        d0.wait(); d1.wait()
        _edge_pass(q, pk_hbm, vals_hbm, h0_v, h1_v, a0_v, a1_v,
                   (p0, p1), (v0, v1), (sem0, sem1), primed)
        e0 = pltpu.async_copy(a0_v, out_hbm.at[pl.ds((q * F + f0) * N, N)], sem2)
        e1 = pltpu.async_copy(a1_v, out_hbm.at[pl.ds((q * F + f0 + 1) * N, N)], sem2)
        e0.wait(); e1.wait()

    return spmm


def _make_spmm_merge():
    @functools.partial(
        pl.kernel,
        out_type=jax.ShapeDtypeStruct((NQ * F * N,), jnp.float32),
        mesh=_mesh(),
        compiler_params=_sc_params,
        scratch_types=[
            pltpu.VMEM((N,), jnp.float32),
            pltpu.VMEM((N,), jnp.float32),
            pltpu.VMEM((N,), jnp.float32),
            pltpu.VMEM((N,), jnp.float32),
            pltpu.VMEM((N,), jnp.float32),
            pltpu.VMEM((N,), jnp.float32),
            pltpu.VMEM((N,), jnp.float32),
            pltpu.VMEM((N,), jnp.float32),
            pltpu.VMEM((N,), jnp.float32),
            pltpu.VMEM((CH,), jnp.int32),
            pltpu.VMEM((CH,), jnp.int32),
            pltpu.VMEM((CH,), jnp.float32),
            pltpu.VMEM((CH,), jnp.float32),
            pltpu.SemaphoreType.DMA,
            pltpu.SemaphoreType.DMA,
            pltpu.SemaphoreType.DMA,
        ],
    )
    def spmm(prev_hbm, f_hbm, pk_hbm, vals_hbm, out_hbm,
             b00, b01, b02, b03, b10, b11, b12, b13, fv_v,
             p0, p1, v0, v1, sem0, sem1, sem2):
        g, q = _tile_ids()
        f0 = 2 * g
        primed = _start_chunk(q, 0, pk_hbm, vals_hbm, p0, v0, sem0)
        ds_ = []
        for qq, dst in ((0, b00), (1, b01), (2, b02), (3, b03)):
            ds_.append(pltpu.async_copy(
                prev_hbm.at[pl.ds((qq * F + f0) * N, N)], dst, sem2))
        for qq, dst in ((0, b10), (1, b11), (2, b12), (3, b13)):
            ds_.append(pltpu.async_copy(
                prev_hbm.at[pl.ds((qq * F + f0 + 1) * N, N)], dst, sem2))
        ds_.append(pltpu.async_copy(f_hbm, fv_v, sem2))
        for d in ds_:
            d.wait()

        @plsc.parallel_loop(0, NVEC, 1, unroll=8)
        def _(i):
            d = pl.ds(i * L, L)
            fv = fv_v[d]
            b00[d] = (b00[d] + b01[d] + b02[d] + b03[d]) * fv
            b10[d] = (b10[d] + b11[d] + b12[d] + b13[d]) * fv

        h0_v, h1_v, a0_v, a1_v = b00, b10, b01, b11
        _zero_plane(a0_v)
        _zero_plane(a1_v)
        _edge_pass(q, pk_hbm, vals_hbm, h0_v, h1_v, a0_v, a1_v,
                   (p0, p1), (v0, v1), (sem0, sem1), primed)
        e0 = pltpu.async_copy(a0_v, out_hbm.at[pl.ds((q * F + f0) * N, N)], sem2)
        e1 = pltpu.async_copy(a1_v, out_hbm.at[pl.ds((q * F + f0 + 1) * N, N)], sem2)
        e0.wait(); e1.wait()

    return spmm


def _make_final_gather(nb):
    half = nb // NC

    @functools.partial(
        pl.kernel,
        out_type=jax.ShapeDtypeStruct((F * nb,), jnp.float32),
        mesh=_mesh(),
        compiler_params=_sc_params,
        scratch_types=[
            pltpu.VMEM((N,), jnp.float32),
            pltpu.VMEM((N,), jnp.float32),
            pltpu.VMEM((N,), jnp.float32),
            pltpu.VMEM((N,), jnp.float32),
            pltpu.VMEM((half,), jnp.int32),
            pltpu.VMEM((half,), jnp.float32),
            pltpu.SemaphoreType.DMA,
        ],
    )
    def fin(prev_hbm, idx_hbm, out_hbm, t0, t1, t2, t3, idx_v, ob_v, sem0):
        c = lax.axis_index("c")
        s = lax.axis_index("s")
        ds_ = [pltpu.async_copy(prev_hbm.at[pl.ds((qq * F + s) * N, N)], dst, sem0)
               for qq, dst in ((0, t0), (1, t1), (2, t2), (3, t3))]
        ds_.append(pltpu.async_copy(idx_hbm.at[pl.ds(c * half, half)], idx_v, sem0))
        for d in ds_:
            d.wait()

        @plsc.parallel_loop(0, NVEC, 1, unroll=8)
        def _(i):
            d = pl.ds(i * L, L)
            t0[d] = t0[d] + t1[d] + t2[d] + t3[d]

        @plsc.parallel_loop(0, half // L, 1, unroll=8)
        def _(i):
            d = pl.ds(i * L, L)
            ob_v[d] = plsc.load_gather(t0, [idx_v[d]])

        pltpu.sync_copy(ob_v, out_hbm.at[pl.ds(s * nb + c * half, half)])

    return fin


def _tc_mid_matmul(parts, W2):
    def body(p_ref, w_ref, o_ref):
        h2 = jax.nn.relu(p_ref[0] + p_ref[1] + p_ref[2] + p_ref[3])
        o_ref[...] = lax.dot_general(
            w_ref[...], h2, (((0,), (0,)), ((), ())),
            preferred_element_type=jnp.float32)

    return pl.pallas_call(
        body,
        out_shape=jax.ShapeDtypeStruct((F, N), jnp.float32),
    )(parts, W2)


def _tc_prep(x, W1, w_idx, i_idx):
    def body(x_ref, w_ref, wi_ref, ii_ref, h_ref, wp_ref, ip_ref):
        xt = x_ref[...].T
        h_ref[...] = lax.dot_general(
            w_ref[...], xt, (((0,), (0,)), ((), ())),
            preferred_element_type=jnp.float32)
        wp_ref[...] = lax.shift_left(wi_ref[0], 16) | wi_ref[1]
        ip_ref[...] = lax.shift_left(ii_ref[0], 16) | ii_ref[1]

    h0t, w_pk, i_pk = pl.pallas_call(
        body,
        out_shape=[
            jax.ShapeDtypeStruct((F, N), jnp.float32),
            jax.ShapeDtypeStruct((E // 128, 128), jnp.int32),
            jax.ShapeDtypeStruct((E // 128, 128), jnp.int32),
        ],
    )(x, W1, w_idx.reshape(2, E // 128, 128), i_idx.reshape(2, E // 128, 128))
    return h0t, w_pk.reshape(-1), i_pk.reshape(-1)


@jax.jit
def kernel(x, wavelet_indices, wavelet_values, inverse_wavelet_indices,
           inverse_wavelet_values, index, W1, f1, W2, f2):
    h0t, w_pk, i_pk = _tc_prep(x, W1, wavelet_indices, inverse_wavelet_indices)
    w_vals = wavelet_values
    i_vals = inverse_wavelet_values

    spmm_plain = _make_spmm_plain()
    spmm_merge = _make_spmm_merge()
    fin = _make_final_gather(index.shape[0])

    nb = index.shape[0]
    pA = spmm_plain(h0t.reshape(-1), i_pk, i_vals)
    pB = spmm_merge(pA, f1, w_pk, w_vals)
    h3t = _tc_mid_matmul(pB.reshape(NQ, F, N), W2)
    pC = spmm_plain(h3t.reshape(-1), i_pk, i_vals)
    pD = spmm_merge(pC, f2, w_pk, w_vals)
    out_t = fin(pD, index)
    return out_t.reshape(F, nb).T


# speedup vs baseline: 1.0726x; 1.0012x over previous
import functools

import jax
import jax.numpy as jnp
from jax import lax
from jax.experimental import pallas as pl
from jax.experimental.pallas import tpu as pltpu
from jax.experimental.pallas import tpu_sc as plsc

N = 10000
IN = 128
F = 16
E = 320000
NC = 2
NS = 16
NG = 8
NQ = 4
EQ = E // NQ
CH = 8000
NCH = EQ // CH
L = 16
NVEC = N // L
CVEC = CH // L

_mesh = lambda: plsc.VectorSubcoreMesh(core_axis_name="c", subcore_axis_name="s")
_sc_params = pltpu.CompilerParams(needs_layout_passes=False)


def _tile_ids():
    c = lax.axis_index("c")
    s = lax.axis_index("s")
    g = lax.rem(s, NG)
    q = c * 2 + lax.div(s, NG)
    return g, q


def _zero_plane(acc_v):
    @plsc.parallel_loop(0, NVEC, 1, unroll=8)
    def _(i):
        acc_v[pl.ds(i * L, L)] = jnp.zeros((L,), jnp.float32)


def _start_chunk(q, k, pk_hbm, vals_hbm, pk_b, vals_b, sem):
    off = q * EQ + k * CH
    return (pltpu.async_copy(pk_hbm.at[pl.ds(off, CH)], pk_b, sem),
            pltpu.async_copy(vals_hbm.at[pl.ds(off, CH)], vals_b, sem))


def _edge_pass(q, pk_hbm, vals_hbm, h0_v, h1_v, a0_v, a1_v,
               pk_v, vals_v, sems, primed):
    def start(k, b):
        return _start_chunk(q, k, pk_hbm, vals_hbm, pk_v[b], vals_v[b], sems[b])

    pending = {0: primed}
    for k in range(NCH):
        b = k % 2
        if k + 1 < NCH:
            pending[k + 1] = start(k + 1, (k + 1) % 2)
        for d in pending.pop(k):
            d.wait()

        pv, vv_ = pk_v[b], vals_v[b]

        @plsc.parallel_loop(0, CVEC, 1, unroll=8)
        def _(i):
            d = pl.ds(i * L, L)
            pk = pv[d]
            ri = lax.shift_right_logical(pk, 16)
            ci = lax.bitwise_and(pk, jnp.int32(0xFFFF))
            vv = vv_[d]
            plsc.addupdate_scatter(a0_v, [ri], plsc.load_gather(h0_v, [ci]) * vv)
            plsc.addupdate_scatter(a1_v, [ri], plsc.load_gather(h1_v, [ci]) * vv)


def _make_spmm_plain():
    @functools.partial(
        pl.kernel,
        out_type=jax.ShapeDtypeStruct((NQ * F * N,), jnp.float32),
        mesh=_mesh(),
        compiler_params=_sc_params,
        scratch_types=[
            pltpu.VMEM((N,), jnp.float32),
            pltpu.VMEM((N,), jnp.float32),
            pltpu.VMEM((N,), jnp.float32),
            pltpu.VMEM((N,), jnp.float32),
            pltpu.VMEM((CH,), jnp.int32),
            pltpu.VMEM((CH,), jnp.int32),
            pltpu.VMEM((CH,), jnp.float32),
            pltpu.VMEM((CH,), jnp.float32),
            pltpu.SemaphoreType.DMA,
            pltpu.SemaphoreType.DMA,
            pltpu.SemaphoreType.DMA,
        ],
    )
    def spmm(h_hbm, pk_hbm, vals_hbm, out_hbm,
             h0_v, h1_v, a0_v, a1_v, p0, p1, v0, v1, sem0, sem1, sem2):
        g, q = _tile_ids()
        f0 = 2 * g
        primed = _start_chunk(q, 0, pk_hbm, vals_hbm, p0, v0, sem0)
        d0 = pltpu.async_copy(h_hbm.at[pl.ds(f0 * N, N)], h0_v, sem2)
        d1 = pltpu.async_copy(h_hbm.at[pl.ds((f0 + 1) * N, N)], h1_v, sem2)
        _zero_plane(a0_v)
        _zero_plane(a1_v)
        d0.wait(); d1.wait()
        _edge_pass(q, pk_hbm, vals_hbm, h0_v, h1_v, a0_v, a1_v,
                   (p0, p1), (v0, v1), (sem0, sem1), primed)
        e0 = pltpu.async_copy(a0_v, out_hbm.at[pl.ds((q * F + f0) * N, N)], sem2)
        e1 = pltpu.async_copy(a1_v, out_hbm.at[pl.ds((q * F + f0 + 1) * N, N)], sem2)
        e0.wait(); e1.wait()

    return spmm


def _make_spmm_merge():
    @functools.partial(
        pl.kernel,
        out_type=jax.ShapeDtypeStruct((NQ * F * N,), jnp.float32),
        mesh=_mesh(),
        compiler_params=_sc_params,
        scratch_types=[
            pltpu.VMEM((N,), jnp.float32),
            pltpu.VMEM((N,), jnp.float32),
            pltpu.VMEM((N,), jnp.float32),
            pltpu.VMEM((N,), jnp.float32),
            pltpu.VMEM((N,), jnp.float32),
            pltpu.VMEM((N,), jnp.float32),
            pltpu.VMEM((N,), jnp.float32),
            pltpu.VMEM((N,), jnp.float32),
            pltpu.VMEM((N,), jnp.float32),
            pltpu.VMEM((CH,), jnp.int32),
            pltpu.VMEM((CH,), jnp.int32),
            pltpu.VMEM((CH,), jnp.float32),
            pltpu.VMEM((CH,), jnp.float32),
            pltpu.SemaphoreType.DMA,
            pltpu.SemaphoreType.DMA,
            pltpu.SemaphoreType.DMA,
        ],
    )
    def spmm(prev_hbm, f_hbm, pk_hbm, vals_hbm, out_hbm,
             b00, b01, b02, b03, b10, b11, b12, b13, fv_v,
             p0, p1, v0, v1, sem0, sem1, sem2):
        g, q = _tile_ids()
        f0 = 2 * g
        primed = _start_chunk(q, 0, pk_hbm, vals_hbm, p0, v0, sem0)
        grp1 = []
        for qq, f, dst in ((0, 0, b00), (1, 0, b01), (0, 1, b10), (1, 1, b11)):
            grp1.append(pltpu.async_copy(
                prev_hbm.at[pl.ds((qq * F + f0 + f) * N, N)], dst, sem2))
        grp2 = []
        for qq, f, dst in ((2, 0, b02), (3, 0, b03), (2, 1, b12), (3, 1, b13)):
            grp2.append(pltpu.async_copy(
                prev_hbm.at[pl.ds((qq * F + f0 + f) * N, N)], dst, sem1))
        grp2.append(pltpu.async_copy(f_hbm, fv_v, sem1))
        for d in grp1:
            d.wait()

        @plsc.parallel_loop(0, NVEC, 1, unroll=8)
        def _(i):
            d = pl.ds(i * L, L)
            b00[d] = b00[d] + b01[d]
            b10[d] = b10[d] + b11[d]

        for d in grp2:
            d.wait()

        @plsc.parallel_loop(0, NVEC, 1, unroll=8)
        def _(i):
            d = pl.ds(i * L, L)
            fv = fv_v[d]
            b00[d] = (b00[d] + b02[d] + b03[d]) * fv
            b10[d] = (b10[d] + b12[d] + b13[d]) * fv

        h0_v, h1_v, a0_v, a1_v = b00, b10, b01, b11
        _zero_plane(a0_v)
        _zero_plane(a1_v)
        _edge_pass(q, pk_hbm, vals_hbm, h0_v, h1_v, a0_v, a1_v,
                   (p0, p1), (v0, v1), (sem0, sem1), primed)
        e0 = pltpu.async_copy(a0_v, out_hbm.at[pl.ds((q * F + f0) * N, N)], sem2)
        e1 = pltpu.async_copy(a1_v, out_hbm.at[pl.ds((q * F + f0 + 1) * N, N)], sem2)
        e0.wait(); e1.wait()

    return spmm


def _make_final_gather(nb):
    half = nb // NC

    @functools.partial(
        pl.kernel,
        out_type=jax.ShapeDtypeStruct((F * nb,), jnp.float32),
        mesh=_mesh(),
        compiler_params=_sc_params,
        scratch_types=[
            pltpu.VMEM((N,), jnp.float32),
            pltpu.VMEM((N,), jnp.float32),
            pltpu.VMEM((N,), jnp.float32),
            pltpu.VMEM((N,), jnp.float32),
            pltpu.VMEM((half,), jnp.int32),
            pltpu.VMEM((half,), jnp.float32),
            pltpu.SemaphoreType.DMA,
        ],
    )
    def fin(prev_hbm, idx_hbm, out_hbm, t0, t1, t2, t3, idx_v, ob_v, sem0):
        c = lax.axis_index("c")
        s = lax.axis_index("s")
        ds_ = [pltpu.async_copy(prev_hbm.at[pl.ds((qq * F + s) * N, N)], dst, sem0)
               for qq, dst in ((0, t0), (1, t1), (2, t2), (3, t3))]
        ds_.append(pltpu.async_copy(idx_hbm.at[pl.ds(c * half, half)], idx_v, sem0))
        for d in ds_:
            d.wait()

        @plsc.parallel_loop(0, NVEC, 1, unroll=8)
        def _(i):
            d = pl.ds(i * L, L)
            t0[d] = t0[d] + t1[d] + t2[d] + t3[d]

        @plsc.parallel_loop(0, half // L, 1, unroll=8)
        def _(i):
            d = pl.ds(i * L, L)
            ob_v[d] = plsc.load_gather(t0, [idx_v[d]])

        pltpu.sync_copy(ob_v, out_hbm.at[pl.ds(s * nb + c * half, half)])

    return fin


def _tc_mid_matmul(parts, W2):
    def body(p_ref, w_ref, o_ref):
        h2 = jax.nn.relu(p_ref[0] + p_ref[1] + p_ref[2] + p_ref[3])
        o_ref[...] = lax.dot_general(
            w_ref[...], h2, (((0,), (0,)), ((), ())),
            preferred_element_type=jnp.float32)

    return pl.pallas_call(
        body,
        out_shape=jax.ShapeDtypeStruct((F, N), jnp.float32),
    )(parts, W2)


def _tc_prep(x, W1, w_idx, i_idx):
    def body(x_ref, w_ref, wi_ref, ii_ref, h_ref, wp_ref, ip_ref):
        xt = x_ref[...].T
        h_ref[...] = lax.dot_general(
            w_ref[...], xt, (((0,), (0,)), ((), ())),
            preferred_element_type=jnp.float32)
        wp_ref[...] = lax.shift_left(wi_ref[0], 16) | wi_ref[1]
        ip_ref[...] = lax.shift_left(ii_ref[0], 16) | ii_ref[1]

    h0t, w_pk, i_pk = pl.pallas_call(
        body,
        out_shape=[
            jax.ShapeDtypeStruct((F, N), jnp.float32),
            jax.ShapeDtypeStruct((E // 128, 128), jnp.int32),
            jax.ShapeDtypeStruct((E // 128, 128), jnp.int32),
        ],
    )(x, W1, w_idx.reshape(2, E // 128, 128), i_idx.reshape(2, E // 128, 128))
    return h0t, w_pk.reshape(-1), i_pk.reshape(-1)


@jax.jit
def kernel(x, wavelet_indices, wavelet_values, inverse_wavelet_indices,
           inverse_wavelet_values, index, W1, f1, W2, f2):
    h0t, w_pk, i_pk = _tc_prep(x, W1, wavelet_indices, inverse_wavelet_indices)
    w_vals = wavelet_values
    i_vals = inverse_wavelet_values

    spmm_plain = _make_spmm_plain()
    spmm_merge = _make_spmm_merge()
    fin = _make_final_gather(index.shape[0])

    nb = index.shape[0]
    pA = spmm_plain(h0t.reshape(-1), i_pk, i_vals)
    pB = spmm_merge(pA, f1, w_pk, w_vals)
    h3t = _tc_mid_matmul(pB.reshape(NQ, F, N), W2)
    pC = spmm_plain(h3t.reshape(-1), i_pk, i_vals)
    pD = spmm_merge(pC, f2, w_pk, w_vals)
    out_t = fin(pD, index)
    return out_t.reshape(F, nb).T
